# baseline (device time: 29145 ns/iter reference)
import jax
import jax.numpy as jnp
from jax import lax
from jax.experimental import pallas as pl
from jax.experimental.pallas import tpu as pltpu

N_DEV = 16


def kernel(x, w_mat):
    m_total, k_shard = x.shape
    n = w_mat.shape[1]
    m_blk = m_total // N_DEV

    def body(x_ref, w_ref, o_ref, xrow_ref, send_sems, recv_sems):
        my = lax.axis_index("i")

        sends = []
        for d in range(1, N_DEV):
            tgt = lax.rem(my + d, N_DEV)
            rdma = pltpu.make_async_remote_copy(
                src_ref=x_ref.at[pl.ds(tgt * m_blk, m_blk), :],
                dst_ref=xrow_ref.at[:, pl.ds(my * k_shard, k_shard)],
                send_sem=send_sems.at[d],
                recv_sem=recv_sems.at[d],
                device_id=(tgt,),
                device_id_type=pl.DeviceIdType.MESH,
            )
            rdma.start()
            sends.append(rdma)

        own = x_ref[pl.ds(my * m_blk, m_blk), :]
        o_ref[...] = jnp.dot(
            own, w_ref[pl.ds(my * k_shard, k_shard), :],
            preferred_element_type=jnp.float32,
        )

        for d in range(1, N_DEV):
            src_dev = lax.rem(my - d + N_DEV, N_DEV)
            recv = pltpu.make_async_remote_copy(
                src_ref=x_ref.at[pl.ds(0, m_blk), :],
                dst_ref=xrow_ref.at[:, pl.ds(src_dev * k_shard, k_shard)],
                send_sem=send_sems.at[0],
                recv_sem=recv_sems.at[d],
                device_id=(my,),
                device_id_type=pl.DeviceIdType.MESH,
            )
            recv.wait_recv()
            chunk = xrow_ref[:, pl.ds(src_dev * k_shard, k_shard)]
            o_ref[...] = o_ref[...] + jnp.dot(
                chunk, w_ref[pl.ds(src_dev * k_shard, k_shard), :],
                preferred_element_type=jnp.float32,
            )

        o_ref[...] = jnp.maximum(o_ref[...], 0.0)

        for rdma in sends:
            rdma.wait_send()

    return pl.pallas_call(
        body,
        out_shape=jax.ShapeDtypeStruct((m_blk, n), jnp.float32),
        in_specs=[
            pl.BlockSpec(memory_space=pltpu.VMEM),
            pl.BlockSpec(memory_space=pltpu.VMEM),
        ],
        out_specs=pl.BlockSpec(memory_space=pltpu.VMEM),
        scratch_shapes=[
            pltpu.VMEM((m_blk, m_total), jnp.float32),
            pltpu.SemaphoreType.DMA((N_DEV,)),
            pltpu.SemaphoreType.DMA((N_DEV,)),
        ],
    )(x, w_mat)


# device time: 25128 ns/iter; 1.1599x vs baseline; 1.1599x over previous
import jax
import jax.numpy as jnp
from jax import lax
from jax.experimental import pallas as pl
from jax.experimental.pallas import tpu as pltpu

N_DEV = 16


def kernel(x, w_mat):
    m_total, k_shard = x.shape
    n = w_mat.shape[1]
    m_blk = m_total // N_DEV

    def body(x_ref, w_ref, o_ref, xrow_ref, send_sems, recv_sems):
        my = lax.axis_index("i")

        with jax.named_scope("barrier"):
            barrier_sem = pltpu.get_barrier_semaphore()
            for d in range(1, N_DEV):
                nbr = lax.rem(my + d, N_DEV)
                pl.semaphore_signal(
                    barrier_sem, inc=1,
                    device_id=(nbr,), device_id_type=pl.DeviceIdType.MESH,
                )
            pl.semaphore_wait(barrier_sem, N_DEV - 1)

        sends = []
        with jax.named_scope("issue_sends"):
            for d in range(1, N_DEV):
                tgt = lax.rem(my + d, N_DEV)
                rdma = pltpu.make_async_remote_copy(
                    src_ref=x_ref.at[pl.ds(tgt * m_blk, m_blk), :],
                    dst_ref=xrow_ref.at[:, pl.ds(my * k_shard, k_shard)],
                    send_sem=send_sems.at[d],
                    recv_sem=recv_sems.at[d],
                    device_id=(tgt,),
                    device_id_type=pl.DeviceIdType.MESH,
                )
                rdma.start()
                sends.append(rdma)

        with jax.named_scope("own_gemm"):
            own = x_ref[pl.ds(my * m_blk, m_blk), :]
            o_ref[...] = jnp.dot(
                own, w_ref[pl.ds(my * k_shard, k_shard), :],
                preferred_element_type=jnp.float32,
            )

        for d in range(1, N_DEV):
            src_dev = lax.rem(my - d + N_DEV, N_DEV)
            with jax.named_scope(f"wait_recv#d={d}"):
                recv = pltpu.make_async_remote_copy(
                    src_ref=x_ref.at[pl.ds(0, m_blk), :],
                    dst_ref=xrow_ref.at[:, pl.ds(src_dev * k_shard, k_shard)],
                    send_sem=send_sems.at[0],
                    recv_sem=recv_sems.at[d],
                    device_id=(my,),
                    device_id_type=pl.DeviceIdType.MESH,
                )
                recv.wait_recv()
            with jax.named_scope(f"gemm#d={d}"):
                chunk = xrow_ref[:, pl.ds(src_dev * k_shard, k_shard)]
                o_ref[...] = o_ref[...] + jnp.dot(
                    chunk, w_ref[pl.ds(src_dev * k_shard, k_shard), :],
                    preferred_element_type=jnp.float32,
                )

        with jax.named_scope("relu"):
            o_ref[...] = jnp.maximum(o_ref[...], 0.0)

        with jax.named_scope("drain_sends"):
            for rdma in sends:
                rdma.wait_send()

    return pl.pallas_call(
        body,
        out_shape=jax.ShapeDtypeStruct((m_blk, n), jnp.float32),
        in_specs=[
            pl.BlockSpec(memory_space=pltpu.VMEM),
            pl.BlockSpec(memory_space=pltpu.VMEM),
        ],
        out_specs=pl.BlockSpec(memory_space=pltpu.VMEM),
        scratch_shapes=[
            pltpu.VMEM((m_blk, m_total), jnp.float32),
            pltpu.SemaphoreType.DMA((N_DEV,)),
            pltpu.SemaphoreType.DMA((N_DEV,)),
        ],
        compiler_params=pltpu.CompilerParams(collective_id=0),
    )(x, w_mat)


# device time: 24053 ns/iter; 1.2117x vs baseline; 1.0447x over previous
import jax
import jax.numpy as jnp
from jax import lax
from jax.experimental import pallas as pl
from jax.experimental.pallas import tpu as pltpu

N_DEV = 16


def kernel(x, w_mat):
    m_total, k_shard = x.shape
    n = w_mat.shape[1]
    m_blk = m_total // N_DEV

    def body(x_ref, w_hbm, o_ref, xrow_ref, w_vmem, send_sems, recv_sems,
             w_sems):
        my = lax.axis_index("i")

        w_copies = []
        for d in range(N_DEV):
            src_dev = lax.rem(my - d + N_DEV, N_DEV)
            cp = pltpu.make_async_copy(
                w_hbm.at[pl.ds(src_dev * k_shard, k_shard), :],
                w_vmem.at[d],
                w_sems.at[d],
            )
            cp.start()
            w_copies.append(cp)

        barrier_sem = pltpu.get_barrier_semaphore()
        for d in range(1, N_DEV):
            nbr = lax.rem(my + d, N_DEV)
            pl.semaphore_signal(
                barrier_sem, inc=1,
                device_id=(nbr,), device_id_type=pl.DeviceIdType.MESH,
            )
        pl.semaphore_wait(barrier_sem, N_DEV - 1)

        sends = []
        for d in range(1, N_DEV):
            tgt = lax.rem(my + d, N_DEV)
            rdma = pltpu.make_async_remote_copy(
                src_ref=x_ref.at[pl.ds(tgt * m_blk, m_blk), :],
                dst_ref=xrow_ref.at[:, pl.ds(my * k_shard, k_shard)],
                send_sem=send_sems.at[d],
                recv_sem=recv_sems.at[d],
                device_id=(tgt,),
                device_id_type=pl.DeviceIdType.MESH,
            )
            rdma.start()
            sends.append(rdma)

        w_copies[0].wait()
        own = x_ref[pl.ds(my * m_blk, m_blk), :]
        o_ref[...] = jnp.dot(
            own, w_vmem[0], preferred_element_type=jnp.float32,
        )

        for d in range(1, N_DEV):
            src_dev = lax.rem(my - d + N_DEV, N_DEV)
            recv = pltpu.make_async_remote_copy(
                src_ref=x_ref.at[pl.ds(0, m_blk), :],
                dst_ref=xrow_ref.at[:, pl.ds(src_dev * k_shard, k_shard)],
                send_sem=send_sems.at[0],
                recv_sem=recv_sems.at[d],
                device_id=(my,),
                device_id_type=pl.DeviceIdType.MESH,
            )
            recv.wait_recv()
            w_copies[d].wait()
            chunk = xrow_ref[:, pl.ds(src_dev * k_shard, k_shard)]
            o_ref[...] = o_ref[...] + jnp.dot(
                chunk, w_vmem[d], preferred_element_type=jnp.float32,
            )

        o_ref[...] = jnp.maximum(o_ref[...], 0.0)

        for rdma in sends:
            rdma.wait_send()

    return pl.pallas_call(
        body,
        out_shape=jax.ShapeDtypeStruct((m_blk, n), jnp.float32),
        in_specs=[
            pl.BlockSpec(memory_space=pltpu.VMEM),
            pl.BlockSpec(memory_space=pl.ANY),
        ],
        out_specs=pl.BlockSpec(memory_space=pltpu.VMEM),
        scratch_shapes=[
            pltpu.VMEM((m_blk, m_total), jnp.float32),
            pltpu.VMEM((N_DEV, k_shard, n), jnp.float32),
            pltpu.SemaphoreType.DMA((N_DEV,)),
            pltpu.SemaphoreType.DMA((N_DEV,)),
            pltpu.SemaphoreType.DMA((N_DEV,)),
        ],
        compiler_params=pltpu.CompilerParams(collective_id=0),
    )(x, w_mat)


# device time: 19635 ns/iter; 1.4843x vs baseline; 1.2250x over previous
import jax
import jax.numpy as jnp
from jax import lax
from jax.experimental import pallas as pl
from jax.experimental.pallas import tpu as pltpu

N_DEV = 16


def kernel(x, w_mat):
    m_total, k_shard = x.shape
    n = w_mat.shape[1]
    m_blk = m_total // N_DEV

    w_mat = pltpu.with_memory_space_constraint(
        w_mat, pltpu.MemorySpace.HBM
    )

    def body(x_ref, w_hbm, o_ref, xrow_ref, w_vmem, send_sems, recv_sems,
             w_sems):
        my = lax.axis_index("i")

        w_copies = []
        for d in range(N_DEV):
            src_dev = lax.rem(my - d + N_DEV, N_DEV)
            cp = pltpu.make_async_copy(
                w_hbm.at[pl.ds(src_dev * k_shard, k_shard), :],
                w_vmem.at[d],
                w_sems.at[d],
            )
            cp.start()
            w_copies.append(cp)

        barrier_sem = pltpu.get_barrier_semaphore()
        for d in range(1, N_DEV):
            nbr = lax.rem(my + d, N_DEV)
            pl.semaphore_signal(
                barrier_sem, inc=1,
                device_id=(nbr,), device_id_type=pl.DeviceIdType.MESH,
            )
        pl.semaphore_wait(barrier_sem, N_DEV - 1)

        sends = []
        for d in range(1, N_DEV):
            tgt = lax.rem(my + d, N_DEV)
            rdma = pltpu.make_async_remote_copy(
                src_ref=x_ref.at[pl.ds(tgt * m_blk, m_blk), :],
                dst_ref=xrow_ref.at[:, pl.ds(my * k_shard, k_shard)],
                send_sem=send_sems.at[d],
                recv_sem=recv_sems.at[d],
                device_id=(tgt,),
                device_id_type=pl.DeviceIdType.MESH,
            )
            rdma.start()
            sends.append(rdma)

        w_copies[0].wait()
        own = x_ref[pl.ds(my * m_blk, m_blk), :]
        o_ref[...] = jnp.dot(
            own, w_vmem[0], preferred_element_type=jnp.float32,
        )

        for d in range(1, N_DEV):
            src_dev = lax.rem(my - d + N_DEV, N_DEV)
            recv = pltpu.make_async_remote_copy(
                src_ref=x_ref.at[pl.ds(0, m_blk), :],
                dst_ref=xrow_ref.at[:, pl.ds(src_dev * k_shard, k_shard)],
                send_sem=send_sems.at[0],
                recv_sem=recv_sems.at[d],
                device_id=(my,),
                device_id_type=pl.DeviceIdType.MESH,
            )
            recv.wait_recv()
            w_copies[d].wait()
            chunk = xrow_ref[:, pl.ds(src_dev * k_shard, k_shard)]
            o_ref[...] = o_ref[...] + jnp.dot(
                chunk, w_vmem[d], preferred_element_type=jnp.float32,
            )

        o_ref[...] = jnp.maximum(o_ref[...], 0.0)

        for rdma in sends:
            rdma.wait_send()

    return pl.pallas_call(
        body,
        out_shape=jax.ShapeDtypeStruct((m_blk, n), jnp.float32),
        in_specs=[
            pl.BlockSpec(memory_space=pltpu.VMEM),
            pl.BlockSpec(memory_space=pl.ANY),
        ],
        out_specs=pl.BlockSpec(memory_space=pltpu.VMEM),
        scratch_shapes=[
            pltpu.VMEM((m_blk, m_total), jnp.float32),
            pltpu.VMEM((N_DEV, k_shard, n), jnp.float32),
            pltpu.SemaphoreType.DMA((N_DEV,)),
            pltpu.SemaphoreType.DMA((N_DEV,)),
            pltpu.SemaphoreType.DMA((N_DEV,)),
        ],
        compiler_params=pltpu.CompilerParams(collective_id=0),
    )(x, w_mat)
